# baseline (speedup 1.0000x reference)
import functools

import jax
import jax.numpy as jnp
from jax import lax
from jax.experimental import pallas as pl
from jax.experimental.pallas import tpu as pltpu
from jax.experimental.pallas import tpu_sc as plsc

_NC = 2
_NS = 16
_NW = _NC * _NS

_G = 128
_K = 4
_CHUNK = _G * _K


def _sc_gather(flat_tables, idx3, rows, h):
    rpw = rows // _NW
    ng = idx3.shape[1]
    nchunks = ng // _K

    mesh = plsc.VectorSubcoreMesh(core_axis_name="c", subcore_axis_name="s")

    @functools.partial(
        pl.kernel,
        mesh=mesh,
        out_type=jax.ShapeDtypeStruct((rows, h), jnp.float32),
        scratch_types=[
            pltpu.VMEM((ng, _G), jnp.int32),
            pltpu.VMEM((_CHUNK, h), jnp.float32),
            pltpu.VMEM((_CHUNK, h), jnp.float32),
            pltpu.SemaphoreType.DMA,
            pltpu.SemaphoreType.DMA,
        ],
        compiler_params=pltpu.CompilerParams(use_tc_tiling_on_sc=False),
    )
    def gather_kernel(tab_hbm, idx_hbm, out_hbm, idx_v, buf0, buf1, sem0, sem1):
        wid = lax.axis_index("s") * _NC + lax.axis_index("c")
        base_row = wid * rpw
        pltpu.sync_copy(idx_hbm.at[wid], idx_v)

        bufs = (buf0, buf1)
        sems = (sem0, sem1)

        def issue(c):
            buf = bufs[c % 2]
            sem = sems[c % 2]
            return [
                pltpu.async_copy(
                    tab_hbm.at[idx_v.at[c * _K + j]],
                    buf.at[pl.ds(j * _G, _G)],
                    sem,
                )
                for j in range(_K)
            ]

        pending = {0: issue(0)}
        for c in range(nchunks):
            if c + 1 < nchunks:
                pending[c + 1] = issue(c + 1)
            for cp in pending.pop(c):
                cp.wait()
            pltpu.sync_copy(
                bufs[c % 2],
                out_hbm.at[pl.ds(base_row + c * _CHUNK, _CHUNK)],
            )

    return gather_kernel(flat_tables, idx3)


def _tc_matmul_relu(emb, w, b2):
    bsz, kd = emb.shape
    h = w.shape[1]
    blk = 1024

    def body(e_ref, w_ref, b_ref, o_ref):
        acc = jnp.dot(e_ref[...], w_ref[...], preferred_element_type=jnp.float32)
        o_ref[...] = jnp.maximum(acc + b_ref[...], 0.0)

    return pl.pallas_call(
        body,
        grid=(bsz // blk,),
        in_specs=[
            pl.BlockSpec((blk, kd), lambda i: (i, 0)),
            pl.BlockSpec((kd, h), lambda i: (0, 0)),
            pl.BlockSpec((1, h), lambda i: (0, 0)),
        ],
        out_specs=pl.BlockSpec((blk, h), lambda i: (i, 0)),
        out_shape=jax.ShapeDtypeStruct((bsz, h), jnp.float32),
    )(emb, w, b2)


def kernel(x, tables, W, b):
    bsz, nf = x.shape
    _, vocab, h = tables.shape
    rows = bsz * nf

    flat_tables = tables.reshape(nf * vocab, h)
    offs = (jnp.arange(nf, dtype=jnp.int32) * vocab)[None, :]
    idx = (x.astype(jnp.int32) + offs).reshape(-1)
    ng = rows // (_NW * _G)
    idx3 = idx.reshape(_NW, ng, _G)

    emb = _sc_gather(flat_tables, idx3, rows, h)
    emb2 = emb.reshape(bsz, nf * h)
    return _tc_matmul_relu(emb2, W, b.reshape(1, h))
